# baseline (speedup 1.0000x reference)
import functools

import jax
import jax.numpy as jnp
from jax import lax
from jax.experimental import pallas as pl
from jax.experimental.pallas import tpu as pltpu
from jax.experimental.pallas import tpu_sc as plsc

N = 10000
D = 128
E = 320000

NC = 2
NS = 16
NW = NC * NS
CH = 128
GIDX = 8
NGRP = 10
NCH = GIDX * NGRP
EPAD = NW * CH * NCH
ACC_ROWS = ((N + 1 + NS * 8 - 1) // (NS * 8)) * (NS * 8)
RPT = ACC_ROWS // NS


def _mesh():
    return plsc.VectorSubcoreMesh(core_axis_name="c", subcore_axis_name="s")


DEG_LEN = ACC_ROWS * 16


@functools.partial(
    pl.kernel,
    out_type=jax.ShapeDtypeStruct((NC * DEG_LEN,), jnp.float32),
    mesh=_mesh(),
    scratch_types=[
        pltpu.VMEM((NCH, CH), jnp.int32),
        pltpu.VMEM((CH,), jnp.float32),
        pltpu.VMEM_SHARED((DEG_LEN,), jnp.float32),
    ],
)
def _deg_kernel(dst16_hbm, zeros_hbm, ones_hbm, out_hbm, idx_all, ones_v, acc):
    cid = lax.axis_index("c")
    sid = lax.axis_index("s")
    wid = cid * NS + sid
    r0 = sid * (RPT * 16)
    pltpu.sync_copy(zeros_hbm.at[pl.ds(r0, RPT * 16)], acc.at[pl.ds(r0, RPT * 16)])
    pltpu.sync_copy(ones_hbm, ones_v)
    pltpu.sync_copy(dst16_hbm.at[wid, :, :], idx_all)
    plsc.subcore_barrier()

    @pl.loop(0, NCH)
    def _(i):
        pltpu.sync_copy(ones_v, acc.at[idx_all.at[i]], add=True)

    plsc.subcore_barrier()
    pltpu.sync_copy(acc.at[pl.ds(r0, RPT * 16)],
                    out_hbm.at[pl.ds(cid * DEG_LEN + r0, RPT * 16)])


@functools.partial(
    pl.kernel,
    out_type=jax.ShapeDtypeStruct((NC, ACC_ROWS, D), jnp.float32),
    mesh=_mesh(),
    scratch_types=[
        pltpu.VMEM((2, GIDX, CH), jnp.int32),
        pltpu.VMEM((2, GIDX, CH), jnp.int32),
        pltpu.VMEM((2, CH, D), jnp.float32),
        pltpu.VMEM_SHARED((ACC_ROWS, D), jnp.float32),
    ] + [pltpu.SemaphoreType.DMA] * 6,
)
def _agg_kernel(src_hbm, dst_hbm, h_hbm, zeros_hbm, out_hbm,
                sidx_g, didx_g, rows, acc, *sems):
    gsem, ssem, isem = sems[0:2], sems[2:4], sems[4:6]
    cid = lax.axis_index("c")
    sid = lax.axis_index("s")
    wid = cid * NS + sid
    r0 = sid * RPT
    pltpu.sync_copy(zeros_hbm.at[pl.ds(r0, RPT)], acc.at[pl.ds(r0, RPT)])
    plsc.subcore_barrier()

    def idx_start(bg, g):
        blk = pl.ds(g * GIDX, GIDX)
        pltpu.async_copy(src_hbm.at[wid, blk, :], sidx_g.at[bg], isem[bg])
        pltpu.async_copy(dst_hbm.at[wid, blk, :], didx_g.at[bg], isem[bg])

    def idx_wait(bg):
        blk = pl.ds(0, GIDX)
        pltpu.make_async_copy(src_hbm.at[wid, blk, :], sidx_g.at[bg],
                              isem[bg]).wait()
        pltpu.make_async_copy(dst_hbm.at[wid, blk, :], didx_g.at[bg],
                              isem[bg]).wait()

    def g_start(b, bg, k):
        pltpu.async_copy(h_hbm.at[sidx_g.at[bg, k]], rows.at[b], gsem[b])

    def g_wait(b):
        pltpu.make_async_copy(h_hbm.at[sidx_g.at[0, 0]], rows.at[b],
                              gsem[b]).wait()

    def s_start(b, bg, k):
        pltpu.async_copy(rows.at[b], acc.at[didx_g.at[bg, k]], ssem[b],
                         add=True)

    def s_wait(b):
        pltpu.make_async_copy(rows.at[b], acc.at[didx_g.at[0, 0]],
                              ssem[b]).wait()

    def group_body(bg, pf_group, first, has_next):
        for k in range(GIDX):
            b = k % 2
            g_wait(b)
            if not (first and k == 0):
                s_wait(1 - b)
            s_start(b, bg, k)
            if k == 2 and pf_group is not None:
                idx_start(1 - bg, pf_group)
            if k < GIDX - 1:
                g_start(1 - b, bg, k + 1)
            elif has_next:
                idx_wait(1 - bg)
                g_start(0, 1 - bg, 0)

    idx_start(0, 0)
    idx_wait(0)
    idx_start(1, 1)
    g_start(0, 0, 0)
    group_body(0, None, first=True, has_next=True)

    @pl.loop(0, (NGRP - 2) // 2)
    def _(m):
        group_body(1, 2 * m + 2, first=False, has_next=True)
        group_body(0, 2 * m + 3, first=False, has_next=True)

    group_body(1, None, first=False, has_next=False)
    s_wait(1)

    plsc.subcore_barrier()
    pltpu.sync_copy(acc.at[pl.ds(r0, RPT)], out_hbm.at[cid, pl.ds(r0, RPT)])


def _h_body(x_ref, w_ref, b_ref, h_ref):
    h = jnp.dot(x_ref[...], w_ref[...], preferred_element_type=jnp.float32)
    h_ref[...] = h + b_ref[...]


_h_call = pl.pallas_call(
    _h_body, out_shape=jax.ShapeDtypeStruct((N, D), jnp.float32))


def _scale_body(h_ref, deg_ref, o_ref):
    deg = 1.0 + deg_ref[0, :N, 0:1] + deg_ref[1, :N, 0:1]
    o_ref[...] = h_ref[...] * lax.rsqrt(deg)


_scale_call = pl.pallas_call(
    _scale_body, out_shape=jax.ShapeDtypeStruct((N, D), jnp.float32))


def _out_body(agg_ref, h_ref, deg_ref, x_ref, g_ref, bt_ref, o_ref):
    deg = 1.0 + deg_ref[0, :N, 0:1] + deg_ref[1, :N, 0:1]
    dinv = lax.rsqrt(deg)
    pre = (agg_ref[0, :N, :] + agg_ref[1, :N, :] + h_ref[...]) * dinv
    mean = jnp.mean(pre, axis=0, keepdims=True)
    cen = pre - mean
    var = jnp.mean(cen * cen, axis=0, keepdims=True)
    y = cen * lax.rsqrt(var + 1e-5) * g_ref[...] + bt_ref[...]
    o_ref[...] = jnp.maximum(y, 0.0) + x_ref[...]


_out_call = pl.pallas_call(
    _out_body, out_shape=jax.ShapeDtypeStruct((N, D), jnp.float32))


def kernel(x, edge_index, W, b, gamma, beta):
    src = edge_index[0].astype(jnp.int32)
    dst = edge_index[1].astype(jnp.int32)
    npad = EPAD - E
    pad_dst = N + jnp.arange(npad, dtype=jnp.int32) % (ACC_ROWS - N)
    src_p = jnp.concatenate([src, jnp.zeros((npad,), jnp.int32)])
    dst_p = jnp.concatenate([dst, pad_dst])
    src3 = src_p.reshape(NW, NCH, CH)
    dst3 = dst_p.reshape(NW, NCH, CH)
    dst16_3 = dst3 * 16
    zeros16 = jnp.zeros((DEG_LEN,), jnp.float32)
    ones16 = jnp.ones((CH,), jnp.float32)
    zerosD = jnp.zeros((ACC_ROWS, D), jnp.float32)

    degacc = _deg_kernel(dst16_3, zeros16, ones16).reshape(NC, ACC_ROWS, 16)
    h0 = _h_call(x, W, b.reshape(1, D))
    h = _scale_call(h0, degacc)
    agg = _agg_kernel(src3, dst3, h, zerosD)
    return _out_call(agg, h, degacc, x,
                     gamma.reshape(1, D), beta.reshape(1, D))


# speedup vs baseline: 2.4740x; 2.4740x over previous
import functools

import jax
import jax.numpy as jnp
from jax import lax
from jax.experimental import pallas as pl
from jax.experimental.pallas import tpu as pltpu
from jax.experimental.pallas import tpu_sc as plsc

N = 10000
D = 128
E = 320000

NC = 2
NS = 16
NW = NC * NS
CH = 128
GIDX = 8
NCH0 = 80
NCH1 = 80
NGRP0 = NCH0 // GIDX
NGRP1 = NCH1 // GIDX
TOTCH = NS * (NCH0 + NCH1)
NCH = (NCH0 + NCH1) // 2
EPAD = TOTCH * CH
ACC_ROWS = ((N + 1 + NS * 8 - 1) // (NS * 8)) * (NS * 8)
RPT = ACC_ROWS // NS


def _mesh():
    return plsc.VectorSubcoreMesh(core_axis_name="c", subcore_axis_name="s")


DEG_LEN = ACC_ROWS * 16


@functools.partial(
    pl.kernel,
    out_type=jax.ShapeDtypeStruct((NC * DEG_LEN,), jnp.float32),
    mesh=_mesh(),
    scratch_types=[
        pltpu.VMEM((NCH, CH), jnp.int32),
        pltpu.VMEM((CH,), jnp.float32),
        pltpu.VMEM_SHARED((DEG_LEN,), jnp.float32),
    ],
)
def _deg_kernel(dst16_hbm, zeros_hbm, ones_hbm, out_hbm, idx_all, ones_v, acc):
    cid = lax.axis_index("c")
    sid = lax.axis_index("s")
    wid = cid * NS + sid
    r0 = sid * (RPT * 16)
    pltpu.sync_copy(zeros_hbm.at[pl.ds(r0, RPT * 16)], acc.at[pl.ds(r0, RPT * 16)])
    pltpu.sync_copy(ones_hbm, ones_v)
    pltpu.sync_copy(dst16_hbm.at[wid, :, :], idx_all)
    plsc.subcore_barrier()

    @pl.loop(0, NCH)
    def _(i):
        pltpu.sync_copy(ones_v, acc.at[idx_all.at[i]], add=True)

    plsc.subcore_barrier()
    pltpu.sync_copy(acc.at[pl.ds(r0, RPT * 16)],
                    out_hbm.at[pl.ds(cid * DEG_LEN + r0, RPT * 16)])


@functools.partial(
    pl.kernel,
    out_type=jax.ShapeDtypeStruct((NC, ACC_ROWS, D), jnp.float32),
    mesh=_mesh(),
    scratch_types=[
        pltpu.VMEM((2, GIDX, CH), jnp.int32),
        pltpu.VMEM((2, GIDX, CH), jnp.int32),
        pltpu.VMEM((2, CH, D), jnp.float32),
        pltpu.VMEM_SHARED((ACC_ROWS, D), jnp.float32),
    ] + [pltpu.SemaphoreType.DMA] * 6,
)
def _agg_kernel(src_hbm, dst_hbm, h_hbm, zeros_hbm, out_hbm,
                sidx_g, didx_g, rows, acc, *sems):
    gsem, ssem, isem = sems[0:2], sems[2:4], sems[4:6]
    cid = lax.axis_index("c")
    sid = lax.axis_index("s")
    r0 = sid * RPT
    pltpu.sync_copy(zeros_hbm.at[pl.ds(r0, RPT)], acc.at[pl.ds(r0, RPT)])
    plsc.subcore_barrier()

    def idx_start(bg, chunk):
        blk = pl.ds(chunk, GIDX)
        pltpu.async_copy(src_hbm.at[blk, :], sidx_g.at[bg], isem[bg])
        pltpu.async_copy(dst_hbm.at[blk, :], didx_g.at[bg], isem[bg])

    def idx_wait(bg):
        blk = pl.ds(0, GIDX)
        pltpu.make_async_copy(src_hbm.at[blk, :], sidx_g.at[bg],
                              isem[bg]).wait()
        pltpu.make_async_copy(dst_hbm.at[blk, :], didx_g.at[bg],
                              isem[bg]).wait()

    H = CH // 2

    def g_start(b, bg, k):
        pltpu.async_copy(h_hbm.at[sidx_g.at[bg, k, pl.ds(0, H)]],
                         rows.at[b, pl.ds(0, H), :], gsem[b])
        pltpu.async_copy(h_hbm.at[sidx_g.at[bg, k, pl.ds(H, H)]],
                         rows.at[b, pl.ds(H, H), :], gsem[b])

    def g_wait(b):
        pltpu.make_async_copy(h_hbm.at[sidx_g.at[0, 0, pl.ds(0, H)]],
                              rows.at[b, pl.ds(0, H), :], gsem[b]).wait()
        pltpu.make_async_copy(h_hbm.at[sidx_g.at[0, 0, pl.ds(0, H)]],
                              rows.at[b, pl.ds(H, H), :], gsem[b]).wait()

    def s_start(b, bg, k):
        pltpu.async_copy(rows.at[b], acc.at[didx_g.at[bg, k]], ssem[b],
                         add=True)

    def s_wait(b):
        pltpu.make_async_copy(rows.at[b], acc.at[didx_g.at[0, 0]],
                              ssem[b]).wait()

    def pipeline(chunk0, ngrp):
        def group_body(bg, pf_chunk, first, has_next):
            for k in range(GIDX):
                b = k % 2
                g_wait(b)
                s_start(b, bg, k)
                if not (first and k == 0):
                    s_wait(1 - b)
                if k == 2 and pf_chunk is not None:
                    idx_start(1 - bg, pf_chunk)
                if k < GIDX - 1:
                    g_start(1 - b, bg, k + 1)
                elif has_next:
                    idx_wait(1 - bg)
                    g_start(0, 1 - bg, 0)

        idx_start(0, chunk0)
        idx_wait(0)
        idx_start(1, chunk0 + GIDX)
        g_start(0, 0, 0)
        group_body(0, None, first=True, has_next=True)

        @pl.loop(0, (ngrp - 2) // 2)
        def _(m):
            group_body(1, chunk0 + (2 * m + 2) * GIDX, first=False,
                       has_next=True)
            group_body(0, chunk0 + (2 * m + 3) * GIDX, first=False,
                       has_next=True)

        group_body(1, None, first=False, has_next=False)
        s_wait(1)

    @pl.when(cid == 0)
    def _():
        pipeline(sid * NCH0, NGRP0)

    @pl.when(cid == 1)
    def _():
        pipeline(NS * NCH0 + sid * NCH1, NGRP1)

    plsc.subcore_barrier()
    pltpu.sync_copy(acc.at[pl.ds(r0, RPT)], out_hbm.at[cid, pl.ds(r0, RPT)])


def _h_body(x_ref, w_ref, b_ref, h_ref):
    h = jnp.dot(x_ref[...], w_ref[...], preferred_element_type=jnp.float32)
    h_ref[...] = h + b_ref[...]


_h_call = pl.pallas_call(
    _h_body, out_shape=jax.ShapeDtypeStruct((N, D), jnp.float32))


def _scale_body(h_ref, deg_ref, o_ref):
    deg = 1.0 + deg_ref[0, :N, 0:1] + deg_ref[1, :N, 0:1]
    o_ref[...] = h_ref[...] * lax.rsqrt(deg)


_scale_call = pl.pallas_call(
    _scale_body, out_shape=jax.ShapeDtypeStruct((N, D), jnp.float32))


def _out_body(agg_ref, h_ref, deg_ref, x_ref, g_ref, bt_ref, o_ref):
    deg = 1.0 + deg_ref[0, :N, 0:1] + deg_ref[1, :N, 0:1]
    dinv = lax.rsqrt(deg)
    pre = (agg_ref[0, :N, :] + agg_ref[1, :N, :] + h_ref[...]) * dinv
    mean = jnp.mean(pre, axis=0, keepdims=True)
    cen = pre - mean
    var = jnp.mean(cen * cen, axis=0, keepdims=True)
    y = cen * lax.rsqrt(var + 1e-5) * g_ref[...] + bt_ref[...]
    o_ref[...] = jnp.maximum(y, 0.0) + x_ref[...]


_out_call = pl.pallas_call(
    _out_body, out_shape=jax.ShapeDtypeStruct((N, D), jnp.float32))


def kernel(x, edge_index, W, b, gamma, beta):
    src = edge_index[0].astype(jnp.int32)
    dst = edge_index[1].astype(jnp.int32)
    npad = EPAD - E
    pad_iota = jnp.arange(npad, dtype=jnp.int32)
    pad_dst = N + pad_iota % (ACC_ROWS - N)
    src_p = jnp.concatenate([src, pad_iota % N])
    dst_p = jnp.concatenate([dst, pad_dst])
    src2 = src_p.reshape(TOTCH, CH)
    dst2 = dst_p.reshape(TOTCH, CH)
    dst16_3 = dst_p.reshape(NW, NCH, CH) * 16
    zeros16 = jnp.zeros((DEG_LEN,), jnp.float32)
    ones16 = jnp.ones((CH,), jnp.float32)
    zerosD = jnp.zeros((ACC_ROWS, D), jnp.float32)

    degacc = _deg_kernel(dst16_3, zeros16, ones16).reshape(NC, ACC_ROWS, 16)
    h0 = _h_call(x, W, b.reshape(1, D))
    h = _scale_call(h0, degacc)
    agg = _agg_kernel(src2, dst2, h, zerosD)
    return _out_call(agg, h, degacc, x,
                     gamma.reshape(1, D), beta.reshape(1, D))


# speedup vs baseline: 2.4761x; 1.0008x over previous
import functools

import jax
import jax.numpy as jnp
from jax import lax
from jax.experimental import pallas as pl
from jax.experimental.pallas import tpu as pltpu
from jax.experimental.pallas import tpu_sc as plsc

N = 10000
D = 128
E = 320000

NC = 2
NS = 16
NW = NC * NS
CH = 128
GIDX = 8
NCH0 = 80
NCH1 = 80
NGRP0 = NCH0 // GIDX
NGRP1 = NCH1 // GIDX
TOTCH = NS * (NCH0 + NCH1)
NCH = (NCH0 + NCH1) // 2
EPAD = TOTCH * CH
ACC_ROWS = ((N + 1 + NS * 8 - 1) // (NS * 8)) * (NS * 8)
RPT = ACC_ROWS // NS


def _mesh():
    return plsc.VectorSubcoreMesh(core_axis_name="c", subcore_axis_name="s")


DEG_LEN = ACC_ROWS * 16


@functools.partial(
    pl.kernel,
    out_type=jax.ShapeDtypeStruct((NC * DEG_LEN,), jnp.float32),
    mesh=_mesh(),
    scratch_types=[
        pltpu.VMEM((NCH, CH), jnp.int32),
        pltpu.VMEM((CH,), jnp.float32),
        pltpu.VMEM_SHARED((DEG_LEN,), jnp.float32),
        pltpu.SemaphoreType.DMA,
    ],
)
def _deg_kernel(dst16_hbm, zeros_hbm, ones_hbm, out_hbm, idx_all, ones_v, acc,
                sem):
    cid = lax.axis_index("c")
    sid = lax.axis_index("s")
    wid = cid * NS + sid
    r0 = sid * (RPT * 16)
    pltpu.sync_copy(zeros_hbm, acc.at[pl.ds(r0, RPT * 16)])
    pltpu.sync_copy(ones_hbm, ones_v)
    pltpu.sync_copy(dst16_hbm.at[wid, :, :], idx_all)
    plsc.subcore_barrier()

    @pl.loop(0, NCH)
    def _(i):
        pltpu.async_copy(ones_v, acc.at[idx_all.at[i]], sem, add=True)

    @pl.loop(0, NCH)
    def _(i):
        pltpu.make_async_copy(ones_v, acc.at[idx_all.at[i]], sem).wait()

    plsc.subcore_barrier()
    pltpu.sync_copy(acc.at[pl.ds(r0, RPT * 16)],
                    out_hbm.at[pl.ds(cid * DEG_LEN + r0, RPT * 16)])


@functools.partial(
    pl.kernel,
    out_type=jax.ShapeDtypeStruct((NC, ACC_ROWS, D), jnp.float32),
    mesh=_mesh(),
    scratch_types=[
        pltpu.VMEM((2, GIDX, CH), jnp.int32),
        pltpu.VMEM((2, GIDX, CH), jnp.int32),
        pltpu.VMEM((2, CH, D), jnp.float32),
        pltpu.VMEM_SHARED((ACC_ROWS, D), jnp.float32),
    ] + [pltpu.SemaphoreType.DMA] * 6,
)
def _agg_kernel(src_hbm, dst_hbm, h_hbm, zeros_hbm, out_hbm,
                sidx_g, didx_g, rows, acc, *sems):
    gsem, ssem, isem = sems[0:2], sems[2:4], sems[4:6]
    cid = lax.axis_index("c")
    sid = lax.axis_index("s")
    r0 = sid * RPT
    pltpu.sync_copy(zeros_hbm, acc.at[pl.ds(r0, RPT)])
    plsc.subcore_barrier()

    def idx_start(bg, chunk):
        blk = pl.ds(chunk, GIDX)
        pltpu.async_copy(src_hbm.at[blk, :], sidx_g.at[bg], isem[bg])
        pltpu.async_copy(dst_hbm.at[blk, :], didx_g.at[bg], isem[bg])

    def idx_wait(bg):
        blk = pl.ds(0, GIDX)
        pltpu.make_async_copy(src_hbm.at[blk, :], sidx_g.at[bg],
                              isem[bg]).wait()
        pltpu.make_async_copy(dst_hbm.at[blk, :], didx_g.at[bg],
                              isem[bg]).wait()

    def g_start(b, bg, k):
        pltpu.async_copy(h_hbm.at[sidx_g.at[bg, k]], rows.at[b], gsem[b])

    def g_wait(b):
        pltpu.make_async_copy(h_hbm.at[sidx_g.at[0, 0]], rows.at[b],
                              gsem[b]).wait()

    def s_start(b, bg, k):
        pltpu.async_copy(rows.at[b], acc.at[didx_g.at[bg, k]], ssem[b],
                         add=True)

    def s_wait(b):
        pltpu.make_async_copy(rows.at[b], acc.at[didx_g.at[0, 0]],
                              ssem[b]).wait()

    def pipeline(chunk0, ngrp):
        def group_body(bg, pf_chunk, first, has_next):
            for k in range(GIDX):
                b = k % 2
                g_wait(b)
                s_start(b, bg, k)
                if not (first and k == 0):
                    s_wait(1 - b)
                if k == 2 and pf_chunk is not None:
                    idx_start(1 - bg, pf_chunk)
                if k < GIDX - 1:
                    g_start(1 - b, bg, k + 1)
                elif has_next:
                    idx_wait(1 - bg)
                    g_start(0, 1 - bg, 0)

        idx_start(0, chunk0)
        idx_wait(0)
        idx_start(1, chunk0 + GIDX)
        g_start(0, 0, 0)
        group_body(0, None, first=True, has_next=True)

        @pl.loop(0, (ngrp - 2) // 2)
        def _(m):
            group_body(1, chunk0 + (2 * m + 2) * GIDX, first=False,
                       has_next=True)
            group_body(0, chunk0 + (2 * m + 3) * GIDX, first=False,
                       has_next=True)

        group_body(1, None, first=False, has_next=False)
        s_wait(1)

    @pl.when(cid == 0)
    def _():
        pipeline(sid * NCH0, NGRP0)

    @pl.when(cid == 1)
    def _():
        pipeline(NS * NCH0 + sid * NCH1, NGRP1)

    plsc.subcore_barrier()
    pltpu.sync_copy(acc.at[pl.ds(r0, RPT)], out_hbm.at[cid, pl.ds(r0, RPT)])


def _h_body(x_ref, w_ref, b_ref, h_ref):
    h = jnp.dot(x_ref[...], w_ref[...], preferred_element_type=jnp.float32)
    h_ref[...] = h + b_ref[...]


_h_call = pl.pallas_call(
    _h_body, out_shape=jax.ShapeDtypeStruct((N, D), jnp.float32))


def _scale_body(h_ref, deg_ref, o_ref):
    deg = 1.0 + deg_ref[0, :N, 0:1] + deg_ref[1, :N, 0:1]
    o_ref[...] = h_ref[...] * lax.rsqrt(deg)


_scale_call = pl.pallas_call(
    _scale_body, out_shape=jax.ShapeDtypeStruct((N, D), jnp.float32))


def _out_body(agg_ref, h_ref, deg_ref, x_ref, g_ref, bt_ref, o_ref):
    deg = 1.0 + deg_ref[0, :N, 0:1] + deg_ref[1, :N, 0:1]
    dinv = lax.rsqrt(deg)
    pre = (agg_ref[0, :N, :] + agg_ref[1, :N, :] + h_ref[...]) * dinv
    mean = jnp.mean(pre, axis=0, keepdims=True)
    cen = pre - mean
    var = jnp.mean(cen * cen, axis=0, keepdims=True)
    y = cen * lax.rsqrt(var + 1e-5) * g_ref[...] + bt_ref[...]
    o_ref[...] = jnp.maximum(y, 0.0) + x_ref[...]


_out_call = pl.pallas_call(
    _out_body, out_shape=jax.ShapeDtypeStruct((N, D), jnp.float32))


def kernel(x, edge_index, W, b, gamma, beta):
    src = edge_index[0].astype(jnp.int32)
    dst = edge_index[1].astype(jnp.int32)
    npad = EPAD - E
    pad_iota = jnp.arange(npad, dtype=jnp.int32)
    pad_dst = N + pad_iota % (ACC_ROWS - N)
    src_p = jnp.concatenate([src, pad_iota % N])
    dst_p = jnp.concatenate([dst, pad_dst])
    src2 = src_p.reshape(TOTCH, CH)
    dst2 = dst_p.reshape(TOTCH, CH)
    dst16_3 = dst_p.reshape(NW, NCH, CH) * 16
    zeros16 = jnp.zeros((RPT * 16,), jnp.float32)
    ones16 = jnp.ones((CH,), jnp.float32)
    zerosD = jnp.zeros((RPT, D), jnp.float32)

    degacc = _deg_kernel(dst16_3, zeros16, ones16).reshape(NC, ACC_ROWS, 16)
    h0 = _h_call(x, W, b.reshape(1, D))
    h = _scale_call(h0, degacc)
    agg = _agg_kernel(src2, dst2, h, zerosD)
    return _out_call(agg, h, degacc, x,
                     gamma.reshape(1, D), beta.reshape(1, D))
